# baseline (speedup 1.0000x reference)
import functools

import jax
import jax.numpy as jnp
from jax import lax
from jax.experimental import pallas as pl
from jax.experimental.pallas import tpu as pltpu
from jax.experimental.pallas import tpu_sc as plsc

F32 = jnp.float32
I32 = jnp.int32

NC = 2
NS = 16
NW = NC * NS
L = 16

H = 128


def _sp(v):
    return jnp.logaddexp(v, 0.0)


def _ssp(v):
    return jnp.logaddexp(v, 0.0) - 0.6931471805599453


def _linear_block(x_ref, w_ref, b_ref, o_ref, *, act):
    v = jnp.dot(x_ref[...], w_ref[...], preferred_element_type=F32) + b_ref[...]
    o_ref[...] = act(v)


def _tc_linear(x, w, b, act, block_rows):
    rows, k = x.shape
    n = w.shape[1]
    grid = rows // block_rows
    return pl.pallas_call(
        functools.partial(_linear_block, act=act),
        grid=(grid,),
        in_specs=[
            pl.BlockSpec((block_rows, k), lambda i: (i, 0)),
            pl.BlockSpec((k, n), lambda i: (0, 0)),
            pl.BlockSpec((1, n), lambda i: (0, 0)),
        ],
        out_specs=pl.BlockSpec((block_rows, n), lambda i: (i, 0)),
        out_shape=jax.ShapeDtypeStruct((rows, n), F32),
    )(x, w, b.reshape(1, n))


def _mlp2_block(x_ref, w1_ref, b1_ref, w2_ref, b2_ref, o_ref, *, act1, act2):
    t = jnp.dot(x_ref[...], w1_ref[...], preferred_element_type=F32) + b1_ref[...]
    t = act1(t)
    v = jnp.dot(t, w2_ref[...], preferred_element_type=F32) + b2_ref[...]
    o_ref[...] = act2(v)


def _tc_mlp2(x, w1, b1, w2, b2, act1, act2, block_rows):
    rows, k = x.shape
    h = w1.shape[1]
    n = w2.shape[1]
    grid = rows // block_rows
    return pl.pallas_call(
        functools.partial(_mlp2_block, act1=act1, act2=act2),
        grid=(grid,),
        in_specs=[
            pl.BlockSpec((block_rows, k), lambda i: (i, 0)),
            pl.BlockSpec((k, h), lambda i: (0, 0)),
            pl.BlockSpec((1, h), lambda i: (0, 0)),
            pl.BlockSpec((h, n), lambda i: (0, 0)),
            pl.BlockSpec((1, n), lambda i: (0, 0)),
        ],
        out_specs=pl.BlockSpec((block_rows, n), lambda i: (i, 0)),
        out_shape=jax.ShapeDtypeStruct((rows, n), F32),
    )(x, w1, b1.reshape(1, h), w2, b2.reshape(1, n))


def _xout_block(agg_ref, w_ref, b_ref, o_ref, *, act):
    a = agg_ref[0] + agg_ref[1]
    o_ref[...] = act(jnp.dot(a, w_ref[...], preferred_element_type=F32)
                     + b_ref[...])


def _tc_xout(agg, w, b, act, block_rows):
    _, rows, k = agg.shape
    n = w.shape[1]
    grid = rows // block_rows
    return pl.pallas_call(
        functools.partial(_xout_block, act=act),
        grid=(grid,),
        in_specs=[
            pl.BlockSpec((2, block_rows, k), lambda i: (0, i, 0)),
            pl.BlockSpec((k, n), lambda i: (0, 0)),
            pl.BlockSpec((1, n), lambda i: (0, 0)),
        ],
        out_specs=pl.BlockSpec((block_rows, n), lambda i: (i, 0)),
        out_shape=jax.ShapeDtypeStruct((rows, n), F32),
    )(agg, w, b.reshape(1, n))


def _hv2_block(y_ref, m_ref, wt_ref, wb_ref, b_ref, o_ref):
    v = jnp.dot(y_ref[...], wt_ref[...], preferred_element_type=F32)
    v = v + jnp.dot(m_ref[...], wb_ref[...], preferred_element_type=F32)
    o_ref[...] = v + b_ref[...]


def _tc_hv2(y, m, wt, wb, b, block_rows):
    rows, k = y.shape
    n = wt.shape[1]
    grid = rows // block_rows
    return pl.pallas_call(
        _hv2_block,
        grid=(grid,),
        in_specs=[
            pl.BlockSpec((block_rows, k), lambda i: (i, 0)),
            pl.BlockSpec((block_rows, H), lambda i: (i, 0)),
            pl.BlockSpec((k, n), lambda i: (0, 0)),
            pl.BlockSpec((H, n), lambda i: (0, 0)),
            pl.BlockSpec((1, n), lambda i: (0, 0)),
        ],
        out_specs=pl.BlockSpec((block_rows, n), lambda i: (i, 0)),
        out_shape=jax.ShapeDtypeStruct((rows, n), F32),
    )(y, m, wt, wb, b.reshape(1, n))


def _g_block(y_ref, m_ref, z_ref, wt_ref, wb_ref, bh_ref,
             w1_ref, b1_ref, w2_ref, b2_ref, o_ref, *, split):
    pid = pl.program_id(0)

    @pl.when(pid < split)
    def _hv2():
        v = jnp.dot(y_ref[...], wt_ref[...], preferred_element_type=F32)
        v = v + jnp.dot(m_ref[...], wb_ref[...], preferred_element_type=F32)
        o_ref[...] = v + bh_ref[...]

    @pl.when(pid >= split)
    def _he2():
        t = _ssp(jnp.dot(z_ref[...], w1_ref[...], preferred_element_type=F32)
                 + b1_ref[...])
        o_ref[...] = _ssp(jnp.dot(t, w2_ref[...], preferred_element_type=F32)
                          + b2_ref[...])


def _tc_g(y, m, z, wt, wb, bh, w1, b1, w2, b2, block_rows):
    e_rows = y.shape[0]
    t_rows = z.shape[0]
    k = y.shape[1]
    split = e_rows // block_rows
    grid = split + t_rows // block_rows
    return pl.pallas_call(
        functools.partial(_g_block, split=split),
        grid=(grid,),
        in_specs=[
            pl.BlockSpec((block_rows, k),
                         lambda i: (jnp.minimum(i, split - 1), 0)),
            pl.BlockSpec((block_rows, H),
                         lambda i: (jnp.minimum(i, split - 1), 0)),
            pl.BlockSpec((block_rows, k),
                         lambda i: (jnp.maximum(i - split, 0), 0)),
            pl.BlockSpec((k, H), lambda i: (0, 0)),
            pl.BlockSpec((H, H), lambda i: (0, 0)),
            pl.BlockSpec((1, H), lambda i: (0, 0)),
            pl.BlockSpec((k, H), lambda i: (0, 0)),
            pl.BlockSpec((1, H), lambda i: (0, 0)),
            pl.BlockSpec((H, H), lambda i: (0, 0)),
            pl.BlockSpec((1, H), lambda i: (0, 0)),
        ],
        out_specs=pl.BlockSpec((block_rows, H), lambda i: (i, 0)),
        out_shape=jax.ShapeDtypeStruct((e_rows + t_rows, H), F32),
    )(y, m, z, wt, wb, bh.reshape(1, H), w1, b1.reshape(1, H),
      w2, b2.reshape(1, H))


def _hvhe_block(x_ref, y_ref, pnw_ref, pnb_ref,
                w1_ref, b1_ref, w2_ref, b2_ref, o_ref, *, split):
    pid = pl.program_id(0)

    @pl.when(pid < split)
    def _hv():
        o_ref[...] = (jnp.dot(x_ref[...], pnw_ref[...],
                              preferred_element_type=F32) + pnb_ref[...])

    @pl.when(pid >= split)
    def _he():
        t = _sp(jnp.dot(y_ref[...], w1_ref[...], preferred_element_type=F32)
                + b1_ref[...])
        o_ref[...] = (jnp.dot(t, w2_ref[...], preferred_element_type=F32)
                      + b2_ref[...])


def _tc_hvhe(x_pad, y_pad, pnw, pnb, w1, b1, w2, b2, block_rows):
    n_rows = x_pad.shape[0]
    e_rows = y_pad.shape[0]
    k = y_pad.shape[1]
    split = n_rows // block_rows
    grid = split + e_rows // block_rows
    return pl.pallas_call(
        functools.partial(_hvhe_block, split=split),
        grid=(grid,),
        in_specs=[
            pl.BlockSpec((block_rows, H),
                         lambda i: (jnp.minimum(i, split - 1), 0)),
            pl.BlockSpec((block_rows, k),
                         lambda i: (jnp.maximum(i - split, 0), 0)),
            pl.BlockSpec((H, H), lambda i: (0, 0)),
            pl.BlockSpec((1, H), lambda i: (0, 0)),
            pl.BlockSpec((k, H), lambda i: (0, 0)),
            pl.BlockSpec((1, H), lambda i: (0, 0)),
            pl.BlockSpec((H, H), lambda i: (0, 0)),
            pl.BlockSpec((1, H), lambda i: (0, 0)),
        ],
        out_specs=pl.BlockSpec((block_rows, H), lambda i: (i, 0)),
        out_shape=jax.ShapeDtypeStruct((n_rows + e_rows, H), F32),
    )(x_pad, y_pad, pnw, pnb.reshape(1, H), w1, b1.reshape(1, H),
      w2, b2.reshape(1, H))


CBE = 64
CB = 128


def _sc_edge_body(n_nodes, heoff, e_pad, g_hbm, src_hbm, dst_hbm,
                  m_hbm, agg_hbm, src_t, dst_t,
                  idx_c0, idx_c1, dst_c0, dst_c1, gb0, gb1, agg_sp,
                  sg0, sg1, sm0, sm1, ss0, ss1):
    ci = lax.axis_index("c")
    s = lax.axis_index("s")
    wid = s * NC + ci
    epw = e_pad // NW
    nchunks = epw // CBE
    share = (n_nodes // (8 * NS)) * 8
    tail = n_nodes + L - share * NS

    pltpu.sync_copy(src_hbm.at[pl.ds(wid * epw, epw)], src_t)
    pltpu.sync_copy(dst_hbm.at[pl.ds(wid * epw, epw)], dst_t)

    def zb(r, _):
        for j in range(H // L):
            gb0[r, pl.ds(j * L, L)] = jnp.zeros((L,), F32)
        return 0
    lax.fori_loop(0, CBE, zb, 0, unroll=2)
    zrows = gb0.at[pl.ds(0, CBE)]
    full = share // CBE
    rem = share - full * CBE
    for t in range(full):
        pltpu.sync_copy(zrows, agg_sp.at[pl.ds(s * share + t * CBE, CBE)])
    if rem:
        pltpu.sync_copy(gb0.at[pl.ds(0, rem)],
                        agg_sp.at[pl.ds(s * share + full * CBE, rem)])

    @pl.when(s == NS - 1)
    def _zero_tail():
        pltpu.sync_copy(gb0.at[pl.ds(0, tail)],
                        agg_sp.at[pl.ds(NS * share, tail)])
    plsc.subcore_barrier()

    slots = ((idx_c0, dst_c0, gb0, sg0, sm0, ss0),
             (idx_c1, dst_c1, gb1, sg1, sm1, ss1))
    iota = lax.iota(I32, L)

    def start(i, b):
        idc, dstc, gb, sg, smm, ssm = slots[b]
        base = wid * epw + i * CBE

        @pl.when(i >= 2)
        def _drain():
            pltpu.make_async_copy(gb.at[pl.ds(CBE, CBE)],
                                  m_hbm.at[pl.ds(base, CBE)], smm).wait()
            pltpu.make_async_copy(gb.at[pl.ds(CBE, CBE)],
                                  agg_sp.at[dstc], ssm).wait()
        lo = i * CBE
        for j in range(CBE // L):
            dd = pl.ds(j * L, L)
            idc[dd] = src_t[pl.ds(lo + j * L, L)]
            idc[pl.ds(CBE + j * L, L)] = (heoff + base + j * L) + iota
            dstc[dd] = dst_t[pl.ds(lo + j * L, L)]
        pltpu.async_copy(g_hbm.at[idc], gb, sg)

    def finish(i, b):
        idc, dstc, gb, sg, smm, ssm = slots[b]
        base = wid * epw + i * CBE
        pltpu.make_async_copy(g_hbm.at[idc], gb, sg).wait()

        def mul(r, _):
            for j in range(H // L):
                sl_ = pl.ds(j * L, L)
                gb[CBE + r, sl_] = gb[CBE + r, sl_] * gb[r, sl_]
            return 0
        lax.fori_loop(0, CBE, mul, 0, unroll=2)

        pltpu.async_copy(gb.at[pl.ds(CBE, CBE)],
                         m_hbm.at[pl.ds(base, CBE)], smm)
        pltpu.async_copy(gb.at[pl.ds(CBE, CBE)], agg_sp.at[dstc], ssm,
                         add=True)

    start(0, 0)

    def pipe(g_, _):
        i1 = 2 * g_ + 1
        start(i1, 1)
        finish(2 * g_, 0)

        @pl.when(2 * g_ + 2 < nchunks)
        def _nxt():
            start(2 * g_ + 2, 0)
        finish(i1, 1)
        return 0
    lax.fori_loop(0, nchunks // 2, pipe, 0)

    pltpu.make_async_copy(gb0.at[pl.ds(CBE, CBE)],
                          m_hbm.at[pl.ds(0, CBE)], sm0).wait()
    pltpu.make_async_copy(gb1.at[pl.ds(CBE, CBE)],
                          m_hbm.at[pl.ds(0, CBE)], sm1).wait()
    pltpu.make_async_copy(gb0.at[pl.ds(CBE, CBE)],
                          agg_sp.at[dst_c0], ss0).wait()
    pltpu.make_async_copy(gb1.at[pl.ds(CBE, CBE)],
                          agg_sp.at[dst_c1], ss1).wait()
    plsc.subcore_barrier()

    pltpu.sync_copy(agg_sp.at[pl.ds(s * share, share)],
                    agg_hbm.at[ci, pl.ds(s * share, share)])
    dtail = n_nodes - NS * share

    @pl.when(s == NS - 1)
    def _drain_tail():
        pltpu.sync_copy(agg_sp.at[pl.ds(NS * share, dtail)],
                        agg_hbm.at[ci, pl.ds(NS * share, dtail)])


def _sc_edge(ghe, n_nodes, heoff, src_pad, dst_pad):
    e_pad = src_pad.shape[0]
    mesh = plsc.VectorSubcoreMesh(core_axis_name="c", subcore_axis_name="s")
    f = pl.kernel(
        functools.partial(_sc_edge_body, n_nodes, heoff, e_pad),
        out_type=(
            jax.ShapeDtypeStruct((e_pad, H), F32),
            jax.ShapeDtypeStruct((NC, n_nodes, H), F32),
        ),
        mesh=mesh,
        scratch_types=[
            pltpu.VMEM((e_pad // NW,), I32),
            pltpu.VMEM((e_pad // NW,), I32),
            pltpu.VMEM((2 * CBE,), I32), pltpu.VMEM((2 * CBE,), I32),
            pltpu.VMEM((CBE,), I32), pltpu.VMEM((CBE,), I32),
            pltpu.VMEM((2 * CBE, H), F32), pltpu.VMEM((2 * CBE, H), F32),
            pltpu.VMEM_SHARED((n_nodes + L, H), F32),
            pltpu.SemaphoreType.DMA, pltpu.SemaphoreType.DMA,
            pltpu.SemaphoreType.DMA, pltpu.SemaphoreType.DMA,
            pltpu.SemaphoreType.DMA, pltpu.SemaphoreType.DMA,
        ],
    )
    return f(ghe, src_pad, dst_pad)


def _bucket_block(ld_ref, pk_ref, cnt_ref, *, rpass, nbkt):
    ld = ld_ref[0]
    bv = ld // rpass
    pk_ref[0] = bv * 8192 + (ld - bv * rpass)
    buckets = lax.broadcasted_iota(I32, (128,), 0)
    m = (bv[:, :, None] == buckets).astype(I32)
    cnt_ref[0] = jnp.sum(m, axis=(0, 1))[None, :]


def _tc_bucket(ldst3, rpass, nbkt):
    nt, rows, cols = ldst3.shape
    return pl.pallas_call(
        functools.partial(_bucket_block, rpass=rpass, nbkt=nbkt),
        grid=(nt,),
        in_specs=[pl.BlockSpec((1, rows, cols), lambda i: (i, 0, 0))],
        out_specs=[
            pl.BlockSpec((1, rows, cols), lambda i: (i, 0, 0)),
            pl.BlockSpec((1, 1, 128), lambda i: (i, 0, 0)),
        ],
        out_shape=[
            jax.ShapeDtypeStruct((nt, rows, cols), I32),
            jax.ShapeDtypeStruct((nt, 1, 128), I32),
        ],
    )(ldst3)


RPASS = 6400
NPASS = 25
NBKT = NPASS + 1
CL = 64
OFF_O, CUR_O, NCH_O, SM_N = 0, 32, 64, 96


def _sc_lg_body(t_pad, n_rows1, arena_n, g_hbm, lsrc_hbm, pk_hbm,
                cnt_hbm, part_hbm, lsrc_t, pk_t, cnt_v,
                lsrc_a, pkd_a, idx_c0, idx_c1, ldst_c0, ldst_c1,
                gb0, gb1, agg_sp, sm, sg0, sg1, ss0, ss1):
    ci = lax.axis_index("c")
    s = lax.axis_index("s")
    wid = s * NC + ci
    tpw = t_pad // NW
    niter = tpw // L
    t0 = wid * tpw
    share = RPASS // NS

    pltpu.sync_copy(lsrc_hbm.at[pl.ds(t0, tpw)], lsrc_t)
    pltpu.sync_copy(pk_hbm.at[pl.ds(t0, tpw)], pk_t)
    pltpu.sync_copy(cnt_hbm.at[pl.ds(wid * 128, 128)], cnt_v)

    zero_v = jnp.zeros((L,), I32)
    dump_v = jnp.full((L,), RPASS, I32)

    def prefill(i, _):
        sl = pl.ds(i * L, L)
        lsrc_a[sl] = zero_v
        pkd_a[sl] = dump_v
        return 0
    lax.fori_loop(0, arena_n // L, prefill, 0, unroll=2)

    cv0 = cnt_v[pl.ds(0, L)]
    cv1 = cnt_v[pl.ds(L, L)]
    off = jnp.int32(0)
    for p in range(NBKT):
        sm[OFF_O + p] = off
        sm[CUR_O + p] = off
        c = cv0[p] if p < L else cv1[p - L]
        sm[NCH_O + p] = (c + CL - 1) // CL
        off = ((off + c + L + CL - 1) // CL) * CL

    def place(i, _):
        pkv = pk_t[pl.ds(i * L, L)]
        svv = lsrc_t[pl.ds(i * L, L)]
        for k in range(L):
            pk = pkv[k]
            b = pk >> 13
            lloc = pk & 8191
            pos = sm[CUR_O + b]
            sm[CUR_O + b] = pos + 1
            pkd_a[pl.ds(pos, L)] = jnp.full((L,), (i * L + k) * 8192 + lloc,
                                            I32)
            lsrc_a[pl.ds(pos, L)] = jnp.full((L,), svv[k], I32)
        return 0
    lax.fori_loop(0, niter, place, 0)

    for p in range(NPASS):
        end = sm[CUR_O + p]
        lsrc_a[pl.ds(end, L)] = zero_v
        pkd_a[pl.ds(end, L)] = dump_v

    lslots = ((idx_c0, ldst_c0, gb0, sg0, ss0),
              (idx_c1, ldst_c1, gb1, sg1, ss1))

    def zero_share():
        def zb(r, _):
            for j in range(H // L):
                gb0[r, pl.ds(j * L, L)] = jnp.zeros((L,), F32)
            return 0
        lax.fori_loop(0, CL, zb, 0, unroll=2)
        zrows = gb0.at[pl.ds(0, CL)]
        for t in range(share // CL):
            pltpu.sync_copy(zrows, agg_sp.at[pl.ds(s * share + t * CL, CL)])
        rem = share - (share // CL) * CL
        if rem:
            pltpu.sync_copy(
                gb0.at[pl.ds(0, rem)],
                agg_sp.at[pl.ds(s * share + (share // CL) * CL, rem)])
        pltpu.sync_copy(gb0.at[pl.ds(0, L)], agg_sp.at[pl.ds(RPASS, L)])

    zero_share()
    plsc.subcore_barrier()

    def one_pass(p, _):
        base = p * RPASS
        offp = sm[OFF_O + p]
        nchp = sm[NCH_O + p]

        def lstart(k, b):
            idc, ldc, gb, sg, ssm = lslots[b]

            @pl.when(k >= 2)
            def _sdrain():
                pltpu.make_async_copy(gb.at[pl.ds(CL, CL)],
                                      agg_sp.at[ldc], ssm).wait()
            cb = offp + k * CL
            for j in range(CL // L):
                ss = pl.ds(cb + j * L, L)
                dd = pl.ds(j * L, L)
                pkv = pkd_a[ss]
                idc[dd] = lsrc_a[ss]
                idc[pl.ds(CL + j * L, L)] = (pkv >> 13) + (t0 + n_rows1)
                ldc[dd] = pkv & 8191
            pltpu.async_copy(g_hbm.at[idc], gb, sg)

        def lfinish(b):
            idc, ldc, gb, sg, ssm = lslots[b]
            pltpu.make_async_copy(g_hbm.at[idc], gb, sg).wait()

            def mul(r, _):
                for j in range(H // L):
                    sl_ = pl.ds(j * L, L)
                    gb[CL + r, sl_] = gb[CL + r, sl_] * gb[r, sl_]
                return 0
            lax.fori_loop(0, CL, mul, 0, unroll=2)

            pltpu.async_copy(gb.at[pl.ds(CL, CL)], agg_sp.at[ldc], ssm,
                             add=True)

        @pl.when(nchp > 0)
        def _prologue():
            lstart(0, 0)

        def pipe(g, _):
            i1 = 2 * g + 1

            @pl.when(i1 < nchp)
            def _s1():
                lstart(i1, 1)
            lfinish(0)

            @pl.when(2 * g + 2 < nchp)
            def _s0():
                lstart(2 * g + 2, 0)

            @pl.when(i1 < nchp)
            def _f1():
                lfinish(1)
            return 0
        lax.fori_loop(0, (nchp + 1) // 2, pipe, 0)

        @pl.when(nchp > 0)
        def _d0():
            pltpu.make_async_copy(gb0.at[pl.ds(CL, CL)],
                                  agg_sp.at[ldst_c0], ss0).wait()

        @pl.when(nchp > 1)
        def _d1():
            pltpu.make_async_copy(gb1.at[pl.ds(CL, CL)],
                                  agg_sp.at[ldst_c1], ss1).wait()
        plsc.subcore_barrier()

        pltpu.sync_copy(agg_sp.at[pl.ds(s * share, share)],
                        part_hbm.at[ci, pl.ds(base + s * share, share)])
        zero_share()
        plsc.subcore_barrier()
        return 0

    lax.fori_loop(0, NPASS, one_pass, 0)


def _sc_lg(g, n_rows1, e_out, lsrc_pad, pk_flat, cnt_flat):
    t_pad = lsrc_pad.shape[0]
    mesh = plsc.VectorSubcoreMesh(core_axis_name="c", subcore_axis_name="s")
    tpw = t_pad // NW
    arena_n = ((tpw + NBKT * (CL + L) + CL - 1) // CL) * CL

    f = pl.kernel(
        functools.partial(_sc_lg_body, t_pad, n_rows1, arena_n),
        out_type=jax.ShapeDtypeStruct((NC, e_out, H), F32),
        mesh=mesh,
        scratch_types=[
            pltpu.VMEM((tpw,), I32),
            pltpu.VMEM((tpw,), I32),
            pltpu.VMEM((128,), I32),
            pltpu.VMEM((arena_n,), I32),
            pltpu.VMEM((arena_n,), I32),
            pltpu.VMEM((2 * CL,), I32), pltpu.VMEM((2 * CL,), I32),
            pltpu.VMEM((CL,), I32), pltpu.VMEM((CL,), I32),
            pltpu.VMEM((2 * CL, H), F32), pltpu.VMEM((2 * CL, H), F32),
            pltpu.VMEM_SHARED((RPASS + L, H), F32),
            pltpu.SMEM((SM_N,), I32),
            pltpu.SemaphoreType.DMA, pltpu.SemaphoreType.DMA,
            pltpu.SemaphoreType.DMA, pltpu.SemaphoreType.DMA,
        ],
    )
    return f(g, lsrc_pad, pk_flat, cnt_flat)


def kernel(x, y, z, edge_index, lg_edge_index,
           pn_w, pn_b, po_w, po_b, pe1_w, pe1_b, pe2_w, pe2_b,
           cf_pn_w, cf_pn_b, cf_pe1_w, cf_pe1_b, cf_pe2_w, cf_pe2_b,
           cf_po_w, cf_po_b):
    n_nodes = x.shape[0]
    n_edges = y.shape[0]
    n_trip = z.shape[0]
    edge_in = y.shape[1]

    e_pad = ((n_edges + NW * CB - 1) // (NW * CB)) * (NW * CB)
    t_pad = ((n_trip + NW * CB - 1) // (NW * CB)) * (NW * CB)

    src = edge_index[0]
    dst = edge_index[1]
    src_pad = jnp.concatenate([src, jnp.zeros((e_pad - n_edges,), I32)])
    dst_pad = jnp.concatenate(
        [dst, jnp.full((e_pad - n_edges,), n_nodes, I32)])
    lsrc_pad = jnp.concatenate(
        [lg_edge_index[0], jnp.zeros((t_pad - n_trip,), I32)])
    ldst_pad = jnp.concatenate(
        [lg_edge_index[1], jnp.full((t_pad - n_trip,), n_edges, I32)])
    y_pad = jnp.concatenate(
        [y, jnp.zeros((e_pad - n_edges, edge_in), F32)])

    heoff = 2 * 8192
    x_pad = jnp.concatenate(
        [x, jnp.zeros((heoff - n_nodes, x.shape[1]), F32)])
    ghe = _tc_hvhe(x_pad, y_pad, pn_w, pn_b, pe1_w, pe1_b, pe2_w, pe2_b,
                   8192)

    m_pad, agg = _sc_edge(ghe, n_nodes, heoff, src_pad, dst_pad)
    m = m_pad[:n_edges]

    tpw = t_pad // NW
    ldst3 = ldst_pad.reshape(NW, tpw // H, H)
    pk3, cnt3 = _tc_bucket(ldst3, RPASS, NBKT)
    pk_flat = pk3.reshape(t_pad)
    cnt_flat = cnt3.reshape(NW * 128)

    g = _tc_g(y, m, z, cf_pn_w[:edge_in], cf_pn_w[edge_in:], cf_pn_b,
              cf_pe1_w, cf_pe1_b, cf_pe2_w, cf_pe2_b, 8000)

    agg2p = _sc_lg(g, n_edges, n_edges, lsrc_pad, pk_flat, cnt_flat)

    x_out = _tc_xout(agg, po_w, po_b, _sp, 2000)
    y_out = _tc_xout(agg2p, cf_po_w, cf_po_b, lambda v: _sp(_ssp(v)), 8000)
    return (x_out, y_out)


# speedup vs baseline: 1.0010x; 1.0010x over previous
import functools

import jax
import jax.numpy as jnp
from jax import lax
from jax.experimental import pallas as pl
from jax.experimental.pallas import tpu as pltpu
from jax.experimental.pallas import tpu_sc as plsc

F32 = jnp.float32
I32 = jnp.int32

NC = 2
NS = 16
NW = NC * NS
L = 16

H = 128


def _sp(v):
    return jnp.logaddexp(v, 0.0)


def _ssp(v):
    return jnp.logaddexp(v, 0.0) - 0.6931471805599453


def _linear_block(x_ref, w_ref, b_ref, o_ref, *, act):
    v = jnp.dot(x_ref[...], w_ref[...], preferred_element_type=F32) + b_ref[...]
    o_ref[...] = act(v)


def _tc_linear(x, w, b, act, block_rows):
    rows, k = x.shape
    n = w.shape[1]
    grid = rows // block_rows
    return pl.pallas_call(
        functools.partial(_linear_block, act=act),
        grid=(grid,),
        in_specs=[
            pl.BlockSpec((block_rows, k), lambda i: (i, 0)),
            pl.BlockSpec((k, n), lambda i: (0, 0)),
            pl.BlockSpec((1, n), lambda i: (0, 0)),
        ],
        out_specs=pl.BlockSpec((block_rows, n), lambda i: (i, 0)),
        out_shape=jax.ShapeDtypeStruct((rows, n), F32),
    )(x, w, b.reshape(1, n))


def _mlp2_block(x_ref, w1_ref, b1_ref, w2_ref, b2_ref, o_ref, *, act1, act2):
    t = jnp.dot(x_ref[...], w1_ref[...], preferred_element_type=F32) + b1_ref[...]
    t = act1(t)
    v = jnp.dot(t, w2_ref[...], preferred_element_type=F32) + b2_ref[...]
    o_ref[...] = act2(v)


def _tc_mlp2(x, w1, b1, w2, b2, act1, act2, block_rows):
    rows, k = x.shape
    h = w1.shape[1]
    n = w2.shape[1]
    grid = rows // block_rows
    return pl.pallas_call(
        functools.partial(_mlp2_block, act1=act1, act2=act2),
        grid=(grid,),
        in_specs=[
            pl.BlockSpec((block_rows, k), lambda i: (i, 0)),
            pl.BlockSpec((k, h), lambda i: (0, 0)),
            pl.BlockSpec((1, h), lambda i: (0, 0)),
            pl.BlockSpec((h, n), lambda i: (0, 0)),
            pl.BlockSpec((1, n), lambda i: (0, 0)),
        ],
        out_specs=pl.BlockSpec((block_rows, n), lambda i: (i, 0)),
        out_shape=jax.ShapeDtypeStruct((rows, n), F32),
    )(x, w1, b1.reshape(1, h), w2, b2.reshape(1, n))


def _xout_block(agg_ref, w_ref, b_ref, o_ref, *, act):
    a = agg_ref[0] + agg_ref[1]
    o_ref[...] = act(jnp.dot(a, w_ref[...], preferred_element_type=F32)
                     + b_ref[...])


def _tc_xout(agg, w, b, act, block_rows):
    _, rows, k = agg.shape
    n = w.shape[1]
    grid = rows // block_rows
    return pl.pallas_call(
        functools.partial(_xout_block, act=act),
        grid=(grid,),
        in_specs=[
            pl.BlockSpec((2, block_rows, k), lambda i: (0, i, 0)),
            pl.BlockSpec((k, n), lambda i: (0, 0)),
            pl.BlockSpec((1, n), lambda i: (0, 0)),
        ],
        out_specs=pl.BlockSpec((block_rows, n), lambda i: (i, 0)),
        out_shape=jax.ShapeDtypeStruct((rows, n), F32),
    )(agg, w, b.reshape(1, n))


def _hv2_block(y_ref, m_ref, wt_ref, wb_ref, b_ref, o_ref):
    v = jnp.dot(y_ref[...], wt_ref[...], preferred_element_type=F32)
    v = v + jnp.dot(m_ref[...], wb_ref[...], preferred_element_type=F32)
    o_ref[...] = v + b_ref[...]


def _tc_hv2(y, m, wt, wb, b, block_rows):
    rows, k = y.shape
    n = wt.shape[1]
    grid = rows // block_rows
    return pl.pallas_call(
        _hv2_block,
        grid=(grid,),
        in_specs=[
            pl.BlockSpec((block_rows, k), lambda i: (i, 0)),
            pl.BlockSpec((block_rows, H), lambda i: (i, 0)),
            pl.BlockSpec((k, n), lambda i: (0, 0)),
            pl.BlockSpec((H, n), lambda i: (0, 0)),
            pl.BlockSpec((1, n), lambda i: (0, 0)),
        ],
        out_specs=pl.BlockSpec((block_rows, n), lambda i: (i, 0)),
        out_shape=jax.ShapeDtypeStruct((rows, n), F32),
    )(y, m, wt, wb, b.reshape(1, n))


def _g_block(y_ref, m_ref, z_ref, wt_ref, wb_ref, bh_ref,
             w1_ref, b1_ref, w2_ref, b2_ref, o_ref, *, split):
    pid = pl.program_id(0)

    @pl.when(pid < split)
    def _hv2():
        v = jnp.dot(y_ref[...], wt_ref[...], preferred_element_type=F32)
        v = v + jnp.dot(m_ref[...], wb_ref[...], preferred_element_type=F32)
        o_ref[...] = v + bh_ref[...]

    @pl.when(pid >= split)
    def _he2():
        t = _ssp(jnp.dot(z_ref[...], w1_ref[...], preferred_element_type=F32)
                 + b1_ref[...])
        o_ref[...] = _ssp(jnp.dot(t, w2_ref[...], preferred_element_type=F32)
                          + b2_ref[...])


def _tc_g(y, m, z, wt, wb, bh, w1, b1, w2, b2, block_rows):
    e_rows = y.shape[0]
    t_rows = z.shape[0]
    k = y.shape[1]
    split = e_rows // block_rows
    grid = split + t_rows // block_rows
    return pl.pallas_call(
        functools.partial(_g_block, split=split),
        grid=(grid,),
        in_specs=[
            pl.BlockSpec((block_rows, k),
                         lambda i: (jnp.minimum(i, split - 1), 0)),
            pl.BlockSpec((block_rows, H),
                         lambda i: (jnp.minimum(i, split - 1), 0)),
            pl.BlockSpec((block_rows, k),
                         lambda i: (jnp.maximum(i - split, 0), 0)),
            pl.BlockSpec((k, H), lambda i: (0, 0)),
            pl.BlockSpec((H, H), lambda i: (0, 0)),
            pl.BlockSpec((1, H), lambda i: (0, 0)),
            pl.BlockSpec((k, H), lambda i: (0, 0)),
            pl.BlockSpec((1, H), lambda i: (0, 0)),
            pl.BlockSpec((H, H), lambda i: (0, 0)),
            pl.BlockSpec((1, H), lambda i: (0, 0)),
        ],
        out_specs=pl.BlockSpec((block_rows, H), lambda i: (i, 0)),
        out_shape=jax.ShapeDtypeStruct((e_rows + t_rows, H), F32),
    )(y, m, z, wt, wb, bh.reshape(1, H), w1, b1.reshape(1, H),
      w2, b2.reshape(1, H))


def _hvhe_block(x_ref, y_ref, pnw_ref, pnb_ref,
                w1_ref, b1_ref, w2_ref, b2_ref, o_ref, *, split):
    pid = pl.program_id(0)

    @pl.when(pid < split)
    def _hv():
        o_ref[...] = (jnp.dot(x_ref[...], pnw_ref[...],
                              preferred_element_type=F32) + pnb_ref[...])

    @pl.when(pid >= split)
    def _he():
        t = _sp(jnp.dot(y_ref[...], w1_ref[...], preferred_element_type=F32)
                + b1_ref[...])
        o_ref[...] = (jnp.dot(t, w2_ref[...], preferred_element_type=F32)
                      + b2_ref[...])


def _tc_hvhe(x_pad, y_pad, pnw, pnb, w1, b1, w2, b2, block_rows):
    n_rows = x_pad.shape[0]
    e_rows = y_pad.shape[0]
    k = y_pad.shape[1]
    split = n_rows // block_rows
    grid = split + e_rows // block_rows
    return pl.pallas_call(
        functools.partial(_hvhe_block, split=split),
        grid=(grid,),
        in_specs=[
            pl.BlockSpec((block_rows, H),
                         lambda i: (jnp.minimum(i, split - 1), 0)),
            pl.BlockSpec((block_rows, k),
                         lambda i: (jnp.maximum(i - split, 0), 0)),
            pl.BlockSpec((H, H), lambda i: (0, 0)),
            pl.BlockSpec((1, H), lambda i: (0, 0)),
            pl.BlockSpec((k, H), lambda i: (0, 0)),
            pl.BlockSpec((1, H), lambda i: (0, 0)),
            pl.BlockSpec((H, H), lambda i: (0, 0)),
            pl.BlockSpec((1, H), lambda i: (0, 0)),
        ],
        out_specs=pl.BlockSpec((block_rows, H), lambda i: (i, 0)),
        out_shape=jax.ShapeDtypeStruct((n_rows + e_rows, H), F32),
    )(x_pad, y_pad, pnw, pnb.reshape(1, H), w1, b1.reshape(1, H),
      w2, b2.reshape(1, H))


CBE = 64
CB = 128


def _sc_edge_body(n_nodes, heoff, e_pad, g_hbm, src_hbm, dst_hbm,
                  m_hbm, agg_hbm, src_t, dst_t,
                  idx_c0, idx_c1, dst_c0, dst_c1, gb0, gb1, agg_sp,
                  sg0, sg1, sl0, sl1, sm0, sm1, ss0, ss1):
    ci = lax.axis_index("c")
    s = lax.axis_index("s")
    wid = s * NC + ci
    epw = e_pad // NW
    nchunks = epw // CBE
    share = (n_nodes // (8 * NS)) * 8
    tail = n_nodes + L - share * NS

    pltpu.sync_copy(src_hbm.at[pl.ds(wid * epw, epw)], src_t)
    pltpu.sync_copy(dst_hbm.at[pl.ds(wid * epw, epw)], dst_t)

    def zb(r, _):
        for j in range(H // L):
            gb0[r, pl.ds(j * L, L)] = jnp.zeros((L,), F32)
        return 0
    lax.fori_loop(0, CBE, zb, 0, unroll=2)
    zrows = gb0.at[pl.ds(0, CBE)]
    full = share // CBE
    rem = share - full * CBE
    for t in range(full):
        pltpu.sync_copy(zrows, agg_sp.at[pl.ds(s * share + t * CBE, CBE)])
    if rem:
        pltpu.sync_copy(gb0.at[pl.ds(0, rem)],
                        agg_sp.at[pl.ds(s * share + full * CBE, rem)])

    @pl.when(s == NS - 1)
    def _zero_tail():
        pltpu.sync_copy(gb0.at[pl.ds(0, tail)],
                        agg_sp.at[pl.ds(NS * share, tail)])
    plsc.subcore_barrier()

    slots = ((idx_c0, dst_c0, gb0, sg0, sl0, sm0, ss0),
             (idx_c1, dst_c1, gb1, sg1, sl1, sm1, ss1))

    def start(i, b):
        idc, dstc, gb, sg, sl, smm, ssm = slots[b]
        base = wid * epw + i * CBE

        @pl.when(i >= 2)
        def _drain():
            pltpu.make_async_copy(gb.at[pl.ds(CBE, CBE)],
                                  m_hbm.at[pl.ds(base, CBE)], smm).wait()
            pltpu.make_async_copy(gb.at[pl.ds(CBE, CBE)],
                                  agg_sp.at[dstc], ssm).wait()
        lo = i * CBE
        for j in range(CBE // L):
            dd = pl.ds(j * L, L)
            idc[dd] = src_t[pl.ds(lo + j * L, L)]
            dstc[dd] = dst_t[pl.ds(lo + j * L, L)]
        pltpu.async_copy(g_hbm.at[idc], gb.at[pl.ds(0, CBE)], sg)
        pltpu.async_copy(g_hbm.at[pl.ds(heoff + base, CBE)],
                         gb.at[pl.ds(CBE, CBE)], sl)

    def finish(i, b):
        idc, dstc, gb, sg, sl, smm, ssm = slots[b]
        base = wid * epw + i * CBE
        pltpu.make_async_copy(g_hbm.at[idc], gb.at[pl.ds(0, CBE)], sg).wait()
        pltpu.make_async_copy(g_hbm.at[pl.ds(heoff + base, CBE)],
                              gb.at[pl.ds(CBE, CBE)], sl).wait()

        def mul(r, _):
            for j in range(H // L):
                sl_ = pl.ds(j * L, L)
                gb[CBE + r, sl_] = gb[CBE + r, sl_] * gb[r, sl_]
            return 0
        lax.fori_loop(0, CBE, mul, 0, unroll=2)

        pltpu.async_copy(gb.at[pl.ds(CBE, CBE)],
                         m_hbm.at[pl.ds(base, CBE)], smm)
        pltpu.async_copy(gb.at[pl.ds(CBE, CBE)], agg_sp.at[dstc], ssm,
                         add=True)

    start(0, 0)

    def pipe(g_, _):
        i1 = 2 * g_ + 1
        start(i1, 1)
        finish(2 * g_, 0)

        @pl.when(2 * g_ + 2 < nchunks)
        def _nxt():
            start(2 * g_ + 2, 0)
        finish(i1, 1)
        return 0
    lax.fori_loop(0, nchunks // 2, pipe, 0)

    pltpu.make_async_copy(gb0.at[pl.ds(CBE, CBE)],
                          m_hbm.at[pl.ds(0, CBE)], sm0).wait()
    pltpu.make_async_copy(gb1.at[pl.ds(CBE, CBE)],
                          m_hbm.at[pl.ds(0, CBE)], sm1).wait()
    pltpu.make_async_copy(gb0.at[pl.ds(CBE, CBE)],
                          agg_sp.at[dst_c0], ss0).wait()
    pltpu.make_async_copy(gb1.at[pl.ds(CBE, CBE)],
                          agg_sp.at[dst_c1], ss1).wait()
    plsc.subcore_barrier()

    pltpu.sync_copy(agg_sp.at[pl.ds(s * share, share)],
                    agg_hbm.at[ci, pl.ds(s * share, share)])
    dtail = n_nodes - NS * share

    @pl.when(s == NS - 1)
    def _drain_tail():
        pltpu.sync_copy(agg_sp.at[pl.ds(NS * share, dtail)],
                        agg_hbm.at[ci, pl.ds(NS * share, dtail)])


def _sc_edge(ghe, n_nodes, heoff, src_pad, dst_pad):
    e_pad = src_pad.shape[0]
    mesh = plsc.VectorSubcoreMesh(core_axis_name="c", subcore_axis_name="s")
    f = pl.kernel(
        functools.partial(_sc_edge_body, n_nodes, heoff, e_pad),
        out_type=(
            jax.ShapeDtypeStruct((e_pad, H), F32),
            jax.ShapeDtypeStruct((NC, n_nodes, H), F32),
        ),
        mesh=mesh,
        scratch_types=[
            pltpu.VMEM((e_pad // NW,), I32),
            pltpu.VMEM((e_pad // NW,), I32),
            pltpu.VMEM((CBE,), I32), pltpu.VMEM((CBE,), I32),
            pltpu.VMEM((CBE,), I32), pltpu.VMEM((CBE,), I32),
            pltpu.VMEM((2 * CBE, H), F32), pltpu.VMEM((2 * CBE, H), F32),
            pltpu.VMEM_SHARED((n_nodes + L, H), F32),
            pltpu.SemaphoreType.DMA, pltpu.SemaphoreType.DMA,
            pltpu.SemaphoreType.DMA, pltpu.SemaphoreType.DMA,
            pltpu.SemaphoreType.DMA, pltpu.SemaphoreType.DMA,
            pltpu.SemaphoreType.DMA, pltpu.SemaphoreType.DMA,
        ],
    )
    return f(ghe, src_pad, dst_pad)


def _bucket_block(ld_ref, pk_ref, cnt_ref, *, rpass, nbkt):
    ld = ld_ref[0]
    bv = ld // rpass
    pk_ref[0] = bv * 8192 + (ld - bv * rpass)
    buckets = lax.broadcasted_iota(I32, (128,), 0)
    m = (bv[:, :, None] == buckets).astype(I32)
    cnt_ref[0] = jnp.sum(m, axis=(0, 1))[None, :]


def _tc_bucket(ldst3, rpass, nbkt):
    nt, rows, cols = ldst3.shape
    return pl.pallas_call(
        functools.partial(_bucket_block, rpass=rpass, nbkt=nbkt),
        grid=(nt,),
        in_specs=[pl.BlockSpec((1, rows, cols), lambda i: (i, 0, 0))],
        out_specs=[
            pl.BlockSpec((1, rows, cols), lambda i: (i, 0, 0)),
            pl.BlockSpec((1, 1, 128), lambda i: (i, 0, 0)),
        ],
        out_shape=[
            jax.ShapeDtypeStruct((nt, rows, cols), I32),
            jax.ShapeDtypeStruct((nt, 1, 128), I32),
        ],
    )(ldst3)


RPASS = 6400
NPASS = 25
NBKT = NPASS + 1
CL = 64
OFF_O, CUR_O, NCH_O, SM_N = 0, 32, 64, 96


def _sc_lg_body(t_pad, n_rows1, arena_n, g_hbm, lsrc_hbm, pk_hbm,
                cnt_hbm, part_hbm, lsrc_t, pk_t, cnt_v,
                lsrc_a, pkd_a, idx_c0, idx_c1, ldst_c0, ldst_c1,
                gb0, gb1, agg_sp, sm, sg0, sg1, ss0, ss1):
    ci = lax.axis_index("c")
    s = lax.axis_index("s")
    wid = s * NC + ci
    tpw = t_pad // NW
    niter = tpw // L
    t0 = wid * tpw
    share = RPASS // NS

    pltpu.sync_copy(lsrc_hbm.at[pl.ds(t0, tpw)], lsrc_t)
    pltpu.sync_copy(pk_hbm.at[pl.ds(t0, tpw)], pk_t)
    pltpu.sync_copy(cnt_hbm.at[pl.ds(wid * 128, 128)], cnt_v)

    zero_v = jnp.zeros((L,), I32)
    dump_v = jnp.full((L,), RPASS, I32)

    def prefill(i, _):
        sl = pl.ds(i * L, L)
        lsrc_a[sl] = zero_v
        pkd_a[sl] = dump_v
        return 0
    lax.fori_loop(0, arena_n // L, prefill, 0, unroll=2)

    cv0 = cnt_v[pl.ds(0, L)]
    cv1 = cnt_v[pl.ds(L, L)]
    off = jnp.int32(0)
    for p in range(NBKT):
        sm[OFF_O + p] = off
        sm[CUR_O + p] = off
        c = cv0[p] if p < L else cv1[p - L]
        sm[NCH_O + p] = (c + CL - 1) // CL
        off = ((off + c + L + CL - 1) // CL) * CL

    def place(i, _):
        pkv = pk_t[pl.ds(i * L, L)]
        svv = lsrc_t[pl.ds(i * L, L)]
        for k in range(L):
            pk = pkv[k]
            b = pk >> 13
            lloc = pk & 8191
            pos = sm[CUR_O + b]
            sm[CUR_O + b] = pos + 1
            pkd_a[pl.ds(pos, L)] = jnp.full((L,), (i * L + k) * 8192 + lloc,
                                            I32)
            lsrc_a[pl.ds(pos, L)] = jnp.full((L,), svv[k], I32)
        return 0
    lax.fori_loop(0, niter, place, 0)

    for p in range(NPASS):
        end = sm[CUR_O + p]
        lsrc_a[pl.ds(end, L)] = zero_v
        pkd_a[pl.ds(end, L)] = dump_v

    lslots = ((idx_c0, ldst_c0, gb0, sg0, ss0),
              (idx_c1, ldst_c1, gb1, sg1, ss1))

    def zero_share():
        def zb(r, _):
            for j in range(H // L):
                gb0[r, pl.ds(j * L, L)] = jnp.zeros((L,), F32)
            return 0
        lax.fori_loop(0, CL, zb, 0, unroll=2)
        zrows = gb0.at[pl.ds(0, CL)]
        for t in range(share // CL):
            pltpu.sync_copy(zrows, agg_sp.at[pl.ds(s * share + t * CL, CL)])
        rem = share - (share // CL) * CL
        if rem:
            pltpu.sync_copy(
                gb0.at[pl.ds(0, rem)],
                agg_sp.at[pl.ds(s * share + (share // CL) * CL, rem)])
        pltpu.sync_copy(gb0.at[pl.ds(0, L)], agg_sp.at[pl.ds(RPASS, L)])

    zero_share()
    plsc.subcore_barrier()

    def one_pass(p, _):
        base = p * RPASS
        offp = sm[OFF_O + p]
        nchp = sm[NCH_O + p]

        def lstart(k, b):
            idc, ldc, gb, sg, ssm = lslots[b]

            @pl.when(k >= 2)
            def _sdrain():
                pltpu.make_async_copy(gb.at[pl.ds(CL, CL)],
                                      agg_sp.at[ldc], ssm).wait()
            cb = offp + k * CL
            for j in range(CL // L):
                ss = pl.ds(cb + j * L, L)
                dd = pl.ds(j * L, L)
                pkv = pkd_a[ss]
                idc[dd] = lsrc_a[ss]
                idc[pl.ds(CL + j * L, L)] = (pkv >> 13) + (t0 + n_rows1)
                ldc[dd] = pkv & 8191
            pltpu.async_copy(g_hbm.at[idc], gb, sg)

        def lfinish(b):
            idc, ldc, gb, sg, ssm = lslots[b]
            pltpu.make_async_copy(g_hbm.at[idc], gb, sg).wait()

            def mul(r, _):
                for j in range(H // L):
                    sl_ = pl.ds(j * L, L)
                    gb[CL + r, sl_] = gb[CL + r, sl_] * gb[r, sl_]
                return 0
            lax.fori_loop(0, CL, mul, 0, unroll=2)

            pltpu.async_copy(gb.at[pl.ds(CL, CL)], agg_sp.at[ldc], ssm,
                             add=True)

        @pl.when(nchp > 0)
        def _prologue():
            lstart(0, 0)

        def pipe(g, _):
            i1 = 2 * g + 1

            @pl.when(i1 < nchp)
            def _s1():
                lstart(i1, 1)
            lfinish(0)

            @pl.when(2 * g + 2 < nchp)
            def _s0():
                lstart(2 * g + 2, 0)

            @pl.when(i1 < nchp)
            def _f1():
                lfinish(1)
            return 0
        lax.fori_loop(0, (nchp + 1) // 2, pipe, 0)

        @pl.when(nchp > 0)
        def _d0():
            pltpu.make_async_copy(gb0.at[pl.ds(CL, CL)],
                                  agg_sp.at[ldst_c0], ss0).wait()

        @pl.when(nchp > 1)
        def _d1():
            pltpu.make_async_copy(gb1.at[pl.ds(CL, CL)],
                                  agg_sp.at[ldst_c1], ss1).wait()
        plsc.subcore_barrier()

        pltpu.sync_copy(agg_sp.at[pl.ds(s * share, share)],
                        part_hbm.at[ci, pl.ds(base + s * share, share)])
        zero_share()
        plsc.subcore_barrier()
        return 0

    lax.fori_loop(0, NPASS, one_pass, 0)


def _sc_lg(g, n_rows1, e_out, lsrc_pad, pk_flat, cnt_flat):
    t_pad = lsrc_pad.shape[0]
    mesh = plsc.VectorSubcoreMesh(core_axis_name="c", subcore_axis_name="s")
    tpw = t_pad // NW
    arena_n = ((tpw + NBKT * (CL + L) + CL - 1) // CL) * CL

    f = pl.kernel(
        functools.partial(_sc_lg_body, t_pad, n_rows1, arena_n),
        out_type=jax.ShapeDtypeStruct((NC, e_out, H), F32),
        mesh=mesh,
        scratch_types=[
            pltpu.VMEM((tpw,), I32),
            pltpu.VMEM((tpw,), I32),
            pltpu.VMEM((128,), I32),
            pltpu.VMEM((arena_n,), I32),
            pltpu.VMEM((arena_n,), I32),
            pltpu.VMEM((2 * CL,), I32), pltpu.VMEM((2 * CL,), I32),
            pltpu.VMEM((CL,), I32), pltpu.VMEM((CL,), I32),
            pltpu.VMEM((2 * CL, H), F32), pltpu.VMEM((2 * CL, H), F32),
            pltpu.VMEM_SHARED((RPASS + L, H), F32),
            pltpu.SMEM((SM_N,), I32),
            pltpu.SemaphoreType.DMA, pltpu.SemaphoreType.DMA,
            pltpu.SemaphoreType.DMA, pltpu.SemaphoreType.DMA,
        ],
    )
    return f(g, lsrc_pad, pk_flat, cnt_flat)


def kernel(x, y, z, edge_index, lg_edge_index,
           pn_w, pn_b, po_w, po_b, pe1_w, pe1_b, pe2_w, pe2_b,
           cf_pn_w, cf_pn_b, cf_pe1_w, cf_pe1_b, cf_pe2_w, cf_pe2_b,
           cf_po_w, cf_po_b):
    n_nodes = x.shape[0]
    n_edges = y.shape[0]
    n_trip = z.shape[0]
    edge_in = y.shape[1]

    e_pad = ((n_edges + NW * CB - 1) // (NW * CB)) * (NW * CB)
    t_pad = ((n_trip + NW * CB - 1) // (NW * CB)) * (NW * CB)

    src = edge_index[0]
    dst = edge_index[1]
    src_pad = jnp.concatenate([src, jnp.zeros((e_pad - n_edges,), I32)])
    dst_pad = jnp.concatenate(
        [dst, jnp.full((e_pad - n_edges,), n_nodes, I32)])
    lsrc_pad = jnp.concatenate(
        [lg_edge_index[0], jnp.zeros((t_pad - n_trip,), I32)])
    ldst_pad = jnp.concatenate(
        [lg_edge_index[1], jnp.full((t_pad - n_trip,), n_edges, I32)])
    y_pad = jnp.concatenate(
        [y, jnp.zeros((e_pad - n_edges, edge_in), F32)])

    heoff = 2 * 8192
    x_pad = jnp.concatenate(
        [x, jnp.zeros((heoff - n_nodes, x.shape[1]), F32)])
    ghe = _tc_hvhe(x_pad, y_pad, pn_w, pn_b, pe1_w, pe1_b, pe2_w, pe2_b,
                   8192)

    m_pad, agg = _sc_edge(ghe, n_nodes, heoff, src_pad, dst_pad)
    m = m_pad[:n_edges]

    tpw = t_pad // NW
    ldst3 = ldst_pad.reshape(NW, tpw // H, H)
    pk3, cnt3 = _tc_bucket(ldst3, RPASS, NBKT)
    pk_flat = pk3.reshape(t_pad)
    cnt_flat = cnt3.reshape(NW * 128)

    g = _tc_g(y, m, z, cf_pn_w[:edge_in], cf_pn_w[edge_in:], cf_pn_b,
              cf_pe1_w, cf_pe1_b, cf_pe2_w, cf_pe2_b, 8000)

    agg2p = _sc_lg(g, n_edges, n_edges, lsrc_pad, pk_flat, cnt_flat)

    x_out = _tc_xout(agg, po_w, po_b, _sp, 2000)
    y_out = _tc_xout(agg2p, cf_po_w, cf_po_b, lambda v: _sp(_ssp(v)), 8000)
    return (x_out, y_out)


# speedup vs baseline: 1.0203x; 1.0193x over previous
import functools

import jax
import jax.numpy as jnp
from jax import lax
from jax.experimental import pallas as pl
from jax.experimental.pallas import tpu as pltpu
from jax.experimental.pallas import tpu_sc as plsc

F32 = jnp.float32
I32 = jnp.int32

NC = 2
NS = 16
NW = NC * NS
L = 16

H = 128


def _sp(v):
    return jnp.logaddexp(v, 0.0)


def _ssp(v):
    return jnp.logaddexp(v, 0.0) - 0.6931471805599453


def _linear_block(x_ref, w_ref, b_ref, o_ref, *, act):
    v = jnp.dot(x_ref[...], w_ref[...], preferred_element_type=F32) + b_ref[...]
    o_ref[...] = act(v)


def _tc_linear(x, w, b, act, block_rows):
    rows, k = x.shape
    n = w.shape[1]
    grid = rows // block_rows
    return pl.pallas_call(
        functools.partial(_linear_block, act=act),
        grid=(grid,),
        in_specs=[
            pl.BlockSpec((block_rows, k), lambda i: (i, 0)),
            pl.BlockSpec((k, n), lambda i: (0, 0)),
            pl.BlockSpec((1, n), lambda i: (0, 0)),
        ],
        out_specs=pl.BlockSpec((block_rows, n), lambda i: (i, 0)),
        out_shape=jax.ShapeDtypeStruct((rows, n), F32),
    )(x, w, b.reshape(1, n))


def _mlp2_block(x_ref, w1_ref, b1_ref, w2_ref, b2_ref, o_ref, *, act1, act2):
    t = jnp.dot(x_ref[...], w1_ref[...], preferred_element_type=F32) + b1_ref[...]
    t = act1(t)
    v = jnp.dot(t, w2_ref[...], preferred_element_type=F32) + b2_ref[...]
    o_ref[...] = act2(v)


def _tc_mlp2(x, w1, b1, w2, b2, act1, act2, block_rows):
    rows, k = x.shape
    h = w1.shape[1]
    n = w2.shape[1]
    grid = rows // block_rows
    return pl.pallas_call(
        functools.partial(_mlp2_block, act1=act1, act2=act2),
        grid=(grid,),
        in_specs=[
            pl.BlockSpec((block_rows, k), lambda i: (i, 0)),
            pl.BlockSpec((k, h), lambda i: (0, 0)),
            pl.BlockSpec((1, h), lambda i: (0, 0)),
            pl.BlockSpec((h, n), lambda i: (0, 0)),
            pl.BlockSpec((1, n), lambda i: (0, 0)),
        ],
        out_specs=pl.BlockSpec((block_rows, n), lambda i: (i, 0)),
        out_shape=jax.ShapeDtypeStruct((rows, n), F32),
    )(x, w1, b1.reshape(1, h), w2, b2.reshape(1, n))


def _xout_block(agg_ref, w_ref, b_ref, o_ref, *, act):
    a = agg_ref[0] + agg_ref[1]
    o_ref[...] = act(jnp.dot(a, w_ref[...], preferred_element_type=F32)
                     + b_ref[...])


def _tc_xout(agg, w, b, act, block_rows):
    _, rows, k = agg.shape
    n = w.shape[1]
    grid = rows // block_rows
    return pl.pallas_call(
        functools.partial(_xout_block, act=act),
        grid=(grid,),
        in_specs=[
            pl.BlockSpec((2, block_rows, k), lambda i: (0, i, 0)),
            pl.BlockSpec((k, n), lambda i: (0, 0)),
            pl.BlockSpec((1, n), lambda i: (0, 0)),
        ],
        out_specs=pl.BlockSpec((block_rows, n), lambda i: (i, 0)),
        out_shape=jax.ShapeDtypeStruct((rows, n), F32),
    )(agg, w, b.reshape(1, n))


def _hv2_block(y_ref, m_ref, wt_ref, wb_ref, b_ref, o_ref):
    v = jnp.dot(y_ref[...], wt_ref[...], preferred_element_type=F32)
    v = v + jnp.dot(m_ref[...], wb_ref[...], preferred_element_type=F32)
    o_ref[...] = v + b_ref[...]


def _tc_hv2(y, m, wt, wb, b, block_rows):
    rows, k = y.shape
    n = wt.shape[1]
    grid = rows // block_rows
    return pl.pallas_call(
        _hv2_block,
        grid=(grid,),
        in_specs=[
            pl.BlockSpec((block_rows, k), lambda i: (i, 0)),
            pl.BlockSpec((block_rows, H), lambda i: (i, 0)),
            pl.BlockSpec((k, n), lambda i: (0, 0)),
            pl.BlockSpec((H, n), lambda i: (0, 0)),
            pl.BlockSpec((1, n), lambda i: (0, 0)),
        ],
        out_specs=pl.BlockSpec((block_rows, n), lambda i: (i, 0)),
        out_shape=jax.ShapeDtypeStruct((rows, n), F32),
    )(y, m, wt, wb, b.reshape(1, n))


def _g_block(y_ref, m_ref, z_ref, wt_ref, wb_ref, bh_ref,
             w1_ref, b1_ref, w2_ref, b2_ref, o_ref, *, split):
    pid = pl.program_id(0)

    @pl.when(pid < split)
    def _hv2():
        v = jnp.dot(y_ref[...], wt_ref[...], preferred_element_type=F32)
        v = v + jnp.dot(m_ref[...], wb_ref[...], preferred_element_type=F32)
        o_ref[...] = v + bh_ref[...]

    @pl.when(pid >= split)
    def _he2():
        t = _ssp(jnp.dot(z_ref[...], w1_ref[...], preferred_element_type=F32)
                 + b1_ref[...])
        o_ref[...] = _ssp(jnp.dot(t, w2_ref[...], preferred_element_type=F32)
                          + b2_ref[...])


def _tc_g(y, m, z, wt, wb, bh, w1, b1, w2, b2, block_rows):
    e_rows = y.shape[0]
    t_rows = z.shape[0]
    k = y.shape[1]
    split = e_rows // block_rows
    grid = split + t_rows // block_rows
    return pl.pallas_call(
        functools.partial(_g_block, split=split),
        grid=(grid,),
        in_specs=[
            pl.BlockSpec((block_rows, k),
                         lambda i: (jnp.minimum(i, split - 1), 0)),
            pl.BlockSpec((block_rows, H),
                         lambda i: (jnp.minimum(i, split - 1), 0)),
            pl.BlockSpec((block_rows, k),
                         lambda i: (jnp.maximum(i - split, 0), 0)),
            pl.BlockSpec((k, H), lambda i: (0, 0)),
            pl.BlockSpec((H, H), lambda i: (0, 0)),
            pl.BlockSpec((1, H), lambda i: (0, 0)),
            pl.BlockSpec((k, H), lambda i: (0, 0)),
            pl.BlockSpec((1, H), lambda i: (0, 0)),
            pl.BlockSpec((H, H), lambda i: (0, 0)),
            pl.BlockSpec((1, H), lambda i: (0, 0)),
        ],
        out_specs=pl.BlockSpec((block_rows, H), lambda i: (i, 0)),
        out_shape=jax.ShapeDtypeStruct((e_rows + t_rows, H), F32),
    )(y, m, z, wt, wb, bh.reshape(1, H), w1, b1.reshape(1, H),
      w2, b2.reshape(1, H))


def _hvhe_block(x_ref, y_ref, pnw_ref, pnb_ref,
                w1_ref, b1_ref, w2_ref, b2_ref, o_ref, *, split):
    pid = pl.program_id(0)

    @pl.when(pid < split)
    def _hv():
        o_ref[...] = (jnp.dot(x_ref[...], pnw_ref[...],
                              preferred_element_type=F32) + pnb_ref[...])

    @pl.when(pid >= split)
    def _he():
        t = _sp(jnp.dot(y_ref[...], w1_ref[...], preferred_element_type=F32)
                + b1_ref[...])
        o_ref[...] = (jnp.dot(t, w2_ref[...], preferred_element_type=F32)
                      + b2_ref[...])


def _tc_hvhe(x_pad, y_pad, pnw, pnb, w1, b1, w2, b2, block_rows):
    n_rows = x_pad.shape[0]
    e_rows = y_pad.shape[0]
    k = y_pad.shape[1]
    split = n_rows // block_rows
    grid = split + e_rows // block_rows
    return pl.pallas_call(
        functools.partial(_hvhe_block, split=split),
        grid=(grid,),
        in_specs=[
            pl.BlockSpec((block_rows, H),
                         lambda i: (jnp.minimum(i, split - 1), 0)),
            pl.BlockSpec((block_rows, k),
                         lambda i: (jnp.maximum(i - split, 0), 0)),
            pl.BlockSpec((H, H), lambda i: (0, 0)),
            pl.BlockSpec((1, H), lambda i: (0, 0)),
            pl.BlockSpec((k, H), lambda i: (0, 0)),
            pl.BlockSpec((1, H), lambda i: (0, 0)),
            pl.BlockSpec((H, H), lambda i: (0, 0)),
            pl.BlockSpec((1, H), lambda i: (0, 0)),
        ],
        out_specs=pl.BlockSpec((block_rows, H), lambda i: (i, 0)),
        out_shape=jax.ShapeDtypeStruct((n_rows + e_rows, H), F32),
    )(x_pad, y_pad, pnw, pnb.reshape(1, H), w1, b1.reshape(1, H),
      w2, b2.reshape(1, H))


CBE = 64
CB = 128


def _sc_edge_body(n_nodes, e_pad, hv_hbm, he_hbm, src_hbm, dst_hbm,
                  m_hbm, agg_hbm, src_t, dst_t,
                  idx_c0, idx_c1, dst_c0, dst_c1, gb0, gb1, agg_sp,
                  sg0, sg1, sl0, sl1, sm0, sm1, ss0, ss1):
    ci = lax.axis_index("c")
    s = lax.axis_index("s")
    wid = s * NC + ci
    epw = e_pad // NW
    nchunks = epw // CBE
    share = (n_nodes // (8 * NS)) * 8
    tail = n_nodes + L - share * NS

    pltpu.sync_copy(src_hbm.at[pl.ds(wid * epw, epw)], src_t)
    pltpu.sync_copy(dst_hbm.at[pl.ds(wid * epw, epw)], dst_t)

    def zb(r, _):
        for j in range(H // L):
            gb0[r, pl.ds(j * L, L)] = jnp.zeros((L,), F32)
        return 0
    lax.fori_loop(0, CBE, zb, 0, unroll=2)
    zrows = gb0.at[pl.ds(0, CBE)]
    full = share // CBE
    rem = share - full * CBE
    for t in range(full):
        pltpu.sync_copy(zrows, agg_sp.at[pl.ds(s * share + t * CBE, CBE)])
    if rem:
        pltpu.sync_copy(gb0.at[pl.ds(0, rem)],
                        agg_sp.at[pl.ds(s * share + full * CBE, rem)])

    @pl.when(s == NS - 1)
    def _zero_tail():
        pltpu.sync_copy(gb0.at[pl.ds(0, tail)],
                        agg_sp.at[pl.ds(NS * share, tail)])
    plsc.subcore_barrier()

    slots = ((idx_c0, dst_c0, gb0, sg0, sl0, sm0, ss0),
             (idx_c1, dst_c1, gb1, sg1, sl1, sm1, ss1))

    def start(i, b):
        idc, dstc, gb, sg, sl, smm, ssm = slots[b]
        base = wid * epw + i * CBE

        @pl.when(i >= 2)
        def _drain():
            pltpu.make_async_copy(gb.at[pl.ds(CBE, CBE)],
                                  m_hbm.at[pl.ds(base, CBE)], smm).wait()
            pltpu.make_async_copy(gb.at[pl.ds(CBE, CBE)],
                                  agg_sp.at[dstc], ssm).wait()
        lo = i * CBE
        for j in range(CBE // L):
            dd = pl.ds(j * L, L)
            idc[dd] = src_t[pl.ds(lo + j * L, L)]
            dstc[dd] = dst_t[pl.ds(lo + j * L, L)]
        pltpu.async_copy(hv_hbm.at[idc], gb.at[pl.ds(0, CBE)], sg)
        pltpu.async_copy(he_hbm.at[pl.ds(base, CBE)],
                         gb.at[pl.ds(CBE, CBE)], sl)

    def finish(i, b):
        idc, dstc, gb, sg, sl, smm, ssm = slots[b]
        base = wid * epw + i * CBE
        pltpu.make_async_copy(hv_hbm.at[idc],
                              gb.at[pl.ds(0, CBE)], sg).wait()
        pltpu.make_async_copy(he_hbm.at[pl.ds(base, CBE)],
                              gb.at[pl.ds(CBE, CBE)], sl).wait()

        def mul(r, _):
            for j in range(H // L):
                sl_ = pl.ds(j * L, L)
                gb[CBE + r, sl_] = gb[CBE + r, sl_] * gb[r, sl_]
            return 0
        lax.fori_loop(0, CBE, mul, 0, unroll=2)

        pltpu.async_copy(gb.at[pl.ds(CBE, CBE)],
                         m_hbm.at[pl.ds(base, CBE)], smm)
        pltpu.async_copy(gb.at[pl.ds(CBE, CBE)], agg_sp.at[dstc], ssm,
                         add=True)

    start(0, 0)

    def pipe(g_, _):
        i1 = 2 * g_ + 1
        start(i1, 1)
        finish(2 * g_, 0)

        @pl.when(2 * g_ + 2 < nchunks)
        def _nxt():
            start(2 * g_ + 2, 0)
        finish(i1, 1)
        return 0
    lax.fori_loop(0, nchunks // 2, pipe, 0)

    pltpu.make_async_copy(gb0.at[pl.ds(CBE, CBE)],
                          m_hbm.at[pl.ds(0, CBE)], sm0).wait()
    pltpu.make_async_copy(gb1.at[pl.ds(CBE, CBE)],
                          m_hbm.at[pl.ds(0, CBE)], sm1).wait()
    pltpu.make_async_copy(gb0.at[pl.ds(CBE, CBE)],
                          agg_sp.at[dst_c0], ss0).wait()
    pltpu.make_async_copy(gb1.at[pl.ds(CBE, CBE)],
                          agg_sp.at[dst_c1], ss1).wait()
    plsc.subcore_barrier()

    pltpu.sync_copy(agg_sp.at[pl.ds(s * share, share)],
                    agg_hbm.at[ci, pl.ds(s * share, share)])
    dtail = n_nodes - NS * share

    @pl.when(s == NS - 1)
    def _drain_tail():
        pltpu.sync_copy(agg_sp.at[pl.ds(NS * share, dtail)],
                        agg_hbm.at[ci, pl.ds(NS * share, dtail)])


def _sc_edge(hv, he_pad, src_pad, dst_pad):
    n_nodes = hv.shape[0]
    e_pad = src_pad.shape[0]
    mesh = plsc.VectorSubcoreMesh(core_axis_name="c", subcore_axis_name="s")
    f = pl.kernel(
        functools.partial(_sc_edge_body, n_nodes, e_pad),
        out_type=(
            jax.ShapeDtypeStruct((e_pad, H), F32),
            jax.ShapeDtypeStruct((NC, n_nodes, H), F32),
        ),
        mesh=mesh,
        scratch_types=[
            pltpu.VMEM((e_pad // NW,), I32),
            pltpu.VMEM((e_pad // NW,), I32),
            pltpu.VMEM((CBE,), I32), pltpu.VMEM((CBE,), I32),
            pltpu.VMEM((CBE,), I32), pltpu.VMEM((CBE,), I32),
            pltpu.VMEM((2 * CBE, H), F32), pltpu.VMEM((2 * CBE, H), F32),
            pltpu.VMEM_SHARED((n_nodes + L, H), F32),
            pltpu.SemaphoreType.DMA, pltpu.SemaphoreType.DMA,
            pltpu.SemaphoreType.DMA, pltpu.SemaphoreType.DMA,
            pltpu.SemaphoreType.DMA, pltpu.SemaphoreType.DMA,
            pltpu.SemaphoreType.DMA, pltpu.SemaphoreType.DMA,
        ],
    )
    return f(hv, he_pad, src_pad, dst_pad)


def _bucket_block(ld_ref, pk_ref, cnt_ref, *, rpass, nbkt):
    ld = ld_ref[0]
    bv = ld // rpass
    pk_ref[0] = bv * 8192 + (ld - bv * rpass)
    buckets = lax.broadcasted_iota(I32, (128,), 0)
    m = (bv[:, :, None] == buckets).astype(I32)
    cnt_ref[0] = jnp.sum(m, axis=(0, 1))[None, :]


def _tc_bucket(ldst3, rpass, nbkt):
    nt, rows, cols = ldst3.shape
    return pl.pallas_call(
        functools.partial(_bucket_block, rpass=rpass, nbkt=nbkt),
        grid=(nt,),
        in_specs=[pl.BlockSpec((1, rows, cols), lambda i: (i, 0, 0))],
        out_specs=[
            pl.BlockSpec((1, rows, cols), lambda i: (i, 0, 0)),
            pl.BlockSpec((1, 1, 128), lambda i: (i, 0, 0)),
        ],
        out_shape=[
            jax.ShapeDtypeStruct((nt, rows, cols), I32),
            jax.ShapeDtypeStruct((nt, 1, 128), I32),
        ],
    )(ldst3)


RPASS = 6400
NPASS = 25
NBKT = NPASS + 1
CL = 64
OFF_O, CUR_O, NCH_O, SM_N = 0, 32, 64, 96


def _sc_lg_body(t_pad, n_rows1, arena_n, g_hbm, lsrc_hbm, pk_hbm,
                cnt_hbm, part_hbm, lsrc_t, pk_t, cnt_v,
                lsrc_a, pkd_a, idx_c0, idx_c1, ldst_c0, ldst_c1,
                gb0, gb1, agg_sp, sm, sg0, sg1, ss0, ss1):
    ci = lax.axis_index("c")
    s = lax.axis_index("s")
    wid = s * NC + ci
    tpw = t_pad // NW
    niter = tpw // L
    t0 = wid * tpw
    share = RPASS // NS

    pltpu.sync_copy(lsrc_hbm.at[pl.ds(t0, tpw)], lsrc_t)
    pltpu.sync_copy(pk_hbm.at[pl.ds(t0, tpw)], pk_t)
    pltpu.sync_copy(cnt_hbm.at[pl.ds(wid * 128, 128)], cnt_v)

    zero_v = jnp.zeros((L,), I32)
    dump_v = jnp.full((L,), RPASS, I32)

    def prefill(i, _):
        sl = pl.ds(i * L, L)
        lsrc_a[sl] = zero_v
        pkd_a[sl] = dump_v
        return 0
    lax.fori_loop(0, arena_n // L, prefill, 0, unroll=2)

    cv0 = cnt_v[pl.ds(0, L)]
    cv1 = cnt_v[pl.ds(L, L)]
    off = jnp.int32(0)
    for p in range(NBKT):
        sm[OFF_O + p] = off
        sm[CUR_O + p] = off
        c = cv0[p] if p < L else cv1[p - L]
        sm[NCH_O + p] = (c + CL - 1) // CL
        off = ((off + c + L + CL - 1) // CL) * CL

    def place(i, _):
        pkv = pk_t[pl.ds(i * L, L)]
        svv = lsrc_t[pl.ds(i * L, L)]
        for k in range(L):
            pk = pkv[k]
            b = pk >> 13
            lloc = pk & 8191
            pos = sm[CUR_O + b]
            sm[CUR_O + b] = pos + 1
            pkd_a[pl.ds(pos, L)] = jnp.full((L,), (i * L + k) * 8192 + lloc,
                                            I32)
            lsrc_a[pl.ds(pos, L)] = jnp.full((L,), svv[k], I32)
        return 0
    lax.fori_loop(0, niter, place, 0)

    for p in range(NPASS):
        end = sm[CUR_O + p]
        lsrc_a[pl.ds(end, L)] = zero_v
        pkd_a[pl.ds(end, L)] = dump_v

    lslots = ((idx_c0, ldst_c0, gb0, sg0, ss0),
              (idx_c1, ldst_c1, gb1, sg1, ss1))

    def zero_share():
        def zb(r, _):
            for j in range(H // L):
                gb0[r, pl.ds(j * L, L)] = jnp.zeros((L,), F32)
            return 0
        lax.fori_loop(0, CL, zb, 0, unroll=2)
        zrows = gb0.at[pl.ds(0, CL)]
        for t in range(share // CL):
            pltpu.sync_copy(zrows, agg_sp.at[pl.ds(s * share + t * CL, CL)])
        rem = share - (share // CL) * CL
        if rem:
            pltpu.sync_copy(
                gb0.at[pl.ds(0, rem)],
                agg_sp.at[pl.ds(s * share + (share // CL) * CL, rem)])
        pltpu.sync_copy(gb0.at[pl.ds(0, L)], agg_sp.at[pl.ds(RPASS, L)])

    zero_share()
    plsc.subcore_barrier()

    def one_pass(p, _):
        base = p * RPASS
        offp = sm[OFF_O + p]
        nchp = sm[NCH_O + p]

        def lstart(k, b):
            idc, ldc, gb, sg, ssm = lslots[b]

            @pl.when(k >= 2)
            def _sdrain():
                pltpu.make_async_copy(gb.at[pl.ds(CL, CL)],
                                      agg_sp.at[ldc], ssm).wait()
            cb = offp + k * CL
            for j in range(CL // L):
                ss = pl.ds(cb + j * L, L)
                dd = pl.ds(j * L, L)
                pkv = pkd_a[ss]
                idc[dd] = lsrc_a[ss]
                idc[pl.ds(CL + j * L, L)] = (pkv >> 13) + (t0 + n_rows1)
                ldc[dd] = pkv & 8191
            pltpu.async_copy(g_hbm.at[idc], gb, sg)

        def lfinish(b):
            idc, ldc, gb, sg, ssm = lslots[b]
            pltpu.make_async_copy(g_hbm.at[idc], gb, sg).wait()

            def mul(r, _):
                for j in range(H // L):
                    sl_ = pl.ds(j * L, L)
                    gb[CL + r, sl_] = gb[CL + r, sl_] * gb[r, sl_]
                return 0
            lax.fori_loop(0, CL, mul, 0, unroll=2)

            pltpu.async_copy(gb.at[pl.ds(CL, CL)], agg_sp.at[ldc], ssm,
                             add=True)

        @pl.when(nchp > 0)
        def _prologue():
            lstart(0, 0)

        def pipe(g, _):
            i1 = 2 * g + 1

            @pl.when(i1 < nchp)
            def _s1():
                lstart(i1, 1)
            lfinish(0)

            @pl.when(2 * g + 2 < nchp)
            def _s0():
                lstart(2 * g + 2, 0)

            @pl.when(i1 < nchp)
            def _f1():
                lfinish(1)
            return 0
        lax.fori_loop(0, (nchp + 1) // 2, pipe, 0)

        @pl.when(nchp > 0)
        def _d0():
            pltpu.make_async_copy(gb0.at[pl.ds(CL, CL)],
                                  agg_sp.at[ldst_c0], ss0).wait()

        @pl.when(nchp > 1)
        def _d1():
            pltpu.make_async_copy(gb1.at[pl.ds(CL, CL)],
                                  agg_sp.at[ldst_c1], ss1).wait()
        plsc.subcore_barrier()

        pltpu.sync_copy(agg_sp.at[pl.ds(s * share, share)],
                        part_hbm.at[ci, pl.ds(base + s * share, share)])
        zero_share()
        plsc.subcore_barrier()
        return 0

    lax.fori_loop(0, NPASS, one_pass, 0)


def _sc_lg(g, n_rows1, e_out, lsrc_pad, pk_flat, cnt_flat):
    t_pad = lsrc_pad.shape[0]
    mesh = plsc.VectorSubcoreMesh(core_axis_name="c", subcore_axis_name="s")
    tpw = t_pad // NW
    arena_n = ((tpw + NBKT * (CL + L) + CL - 1) // CL) * CL

    f = pl.kernel(
        functools.partial(_sc_lg_body, t_pad, n_rows1, arena_n),
        out_type=jax.ShapeDtypeStruct((NC, e_out, H), F32),
        mesh=mesh,
        scratch_types=[
            pltpu.VMEM((tpw,), I32),
            pltpu.VMEM((tpw,), I32),
            pltpu.VMEM((128,), I32),
            pltpu.VMEM((arena_n,), I32),
            pltpu.VMEM((arena_n,), I32),
            pltpu.VMEM((2 * CL,), I32), pltpu.VMEM((2 * CL,), I32),
            pltpu.VMEM((CL,), I32), pltpu.VMEM((CL,), I32),
            pltpu.VMEM((2 * CL, H), F32), pltpu.VMEM((2 * CL, H), F32),
            pltpu.VMEM_SHARED((RPASS + L, H), F32),
            pltpu.SMEM((SM_N,), I32),
            pltpu.SemaphoreType.DMA, pltpu.SemaphoreType.DMA,
            pltpu.SemaphoreType.DMA, pltpu.SemaphoreType.DMA,
        ],
    )
    return f(g, lsrc_pad, pk_flat, cnt_flat)


def kernel(x, y, z, edge_index, lg_edge_index,
           pn_w, pn_b, po_w, po_b, pe1_w, pe1_b, pe2_w, pe2_b,
           cf_pn_w, cf_pn_b, cf_pe1_w, cf_pe1_b, cf_pe2_w, cf_pe2_b,
           cf_po_w, cf_po_b):
    n_nodes = x.shape[0]
    n_edges = y.shape[0]
    n_trip = z.shape[0]
    edge_in = y.shape[1]

    e_pad = ((n_edges + NW * CB - 1) // (NW * CB)) * (NW * CB)
    t_pad = ((n_trip + NW * CB - 1) // (NW * CB)) * (NW * CB)

    src = edge_index[0]
    dst = edge_index[1]
    src_pad = jnp.concatenate([src, jnp.zeros((e_pad - n_edges,), I32)])
    dst_pad = jnp.concatenate(
        [dst, jnp.full((e_pad - n_edges,), n_nodes, I32)])
    lsrc_pad = jnp.concatenate(
        [lg_edge_index[0], jnp.zeros((t_pad - n_trip,), I32)])
    ldst_pad = jnp.concatenate(
        [lg_edge_index[1], jnp.full((t_pad - n_trip,), n_edges, I32)])
    y_pad = jnp.concatenate(
        [y, jnp.zeros((e_pad - n_edges, edge_in), F32)])

    hv = _tc_linear(x, pn_w, pn_b, lambda v: v, 2000)
    he_pad = _tc_mlp2(y_pad, pe1_w, pe1_b, pe2_w, pe2_b,
                      _sp, lambda v: v, 8192)

    m_pad, agg = _sc_edge(hv, he_pad, src_pad, dst_pad)
    m = m_pad[:n_edges]

    tpw = t_pad // NW
    ldst3 = ldst_pad.reshape(NW, tpw // H, H)
    pk3, cnt3 = _tc_bucket(ldst3, RPASS, NBKT)
    pk_flat = pk3.reshape(t_pad)
    cnt_flat = cnt3.reshape(NW * 128)

    g = _tc_g(y, m, z, cf_pn_w[:edge_in], cf_pn_w[edge_in:], cf_pn_b,
              cf_pe1_w, cf_pe1_b, cf_pe2_w, cf_pe2_b, 8000)

    agg2p = _sc_lg(g, n_edges, n_edges, lsrc_pad, pk_flat, cnt_flat)

    x_out = _tc_xout(agg, po_w, po_b, _sp, 2000)
    y_out = _tc_xout(agg2p, cf_po_w, cf_po_b, lambda v: _sp(_ssp(v)), 8000)
    return (x_out, y_out)
